# baseline (speedup 1.0000x reference)
import jax
import jax.numpy as jnp
from jax import lax
from jax.experimental import pallas as pl
from jax.experimental.pallas import tpu as pltpu

_NBUF = 2


def _fused_kernel(mask_ref, vals_ref, out_ref, idx_vmem, idx_smem, lv, rep,
                  isem, gsem, sems):
    B, L = mask_ref.shape
    D = lv.shape[-1]

    pos = lax.broadcasted_iota(jnp.int32, (B, L), 1)
    m = mask_ref[...].astype(jnp.int32)
    cand = jnp.where(m == 0, pos, -1)
    idx_vmem[...] = jnp.maximum(jnp.max(cand, axis=1), 0)
    cp = pltpu.make_async_copy(idx_vmem, idx_smem, isem)
    cp.start()
    cp.wait()

    def fire(b, _):
        pltpu.make_async_copy(
            vals_ref.at[b, idx_smem[b]], lv.at[b, 0], gsem
        ).start()
        return 0

    lax.fori_loop(0, B, fire, 0)

    def body(b, _):
        pltpu.make_async_copy(vals_ref.at[b, 0], lv.at[b, 0], gsem).wait()
        j = lax.rem(b, _NBUF)

        @pl.when(b >= _NBUF)
        def _():
            pltpu.make_async_copy(
                rep.at[j], out_ref.at[b - _NBUF], sems.at[j]
            ).wait()

        row = lv[pl.ds(b, 1), 0, :]
        rep[pl.ds(j, 1)] = jnp.broadcast_to(row[None], (1, L, D))
        pltpu.make_async_copy(rep.at[j], out_ref.at[b], sems.at[j]).start()
        return 0

    lax.fori_loop(0, B, body, 0)

    def drain(k, _):
        b = B - _NBUF + k
        pltpu.make_async_copy(
            rep.at[lax.rem(b, _NBUF)], out_ref.at[b], sems.at[lax.rem(b, _NBUF)]
        ).wait()
        return 0

    lax.fori_loop(0, _NBUF, drain, 0)


def kernel(input_values, input_timestamps, is_target_mask, dummy):
    B, L, D = input_values.shape
    mask_i8 = is_target_mask.view(jnp.int8)

    out = pl.pallas_call(
        _fused_kernel,
        in_specs=[
            pl.BlockSpec(memory_space=pltpu.VMEM),
            pl.BlockSpec(memory_space=pl.ANY),
        ],
        out_specs=pl.BlockSpec(memory_space=pl.ANY),
        scratch_shapes=[
            pltpu.VMEM((B,), jnp.int32),
            pltpu.SMEM((B,), jnp.int32),
            pltpu.VMEM((B, 1, D), jnp.float32),
            pltpu.VMEM((_NBUF, L, D), jnp.float32),
            pltpu.SemaphoreType.DMA,
            pltpu.SemaphoreType.DMA,
            pltpu.SemaphoreType.DMA((_NBUF,)),
        ],
        out_shape=jax.ShapeDtypeStruct((B, L, D), jnp.float32),
    )(mask_i8, input_values)
    return out


# speedup vs baseline: 1.6316x; 1.6316x over previous
import jax
import jax.numpy as jnp
from jax import lax
from jax.experimental import pallas as pl
from jax.experimental.pallas import tpu as pltpu

_NBUF = 4
_RP = 2


def _fused_kernel(mask_ref, vals_ref, out_ref, idx_vmem, idx_smem, lv, rep,
                  isem, gsem, sems):
    B, L = mask_ref.shape
    D = lv.shape[-1]

    pos = lax.broadcasted_iota(jnp.int32, (B, L), 1)
    m = mask_ref[...].astype(jnp.int32)
    cand = jnp.where(m == 0, pos, -1)
    idx_vmem[...] = jnp.maximum(jnp.max(cand, axis=1), 0)
    cp = pltpu.make_async_copy(idx_vmem, idx_smem, isem)
    cp.start()
    cp.wait()

    def fire(b, _):
        pltpu.make_async_copy(
            vals_ref.at[b, idx_smem[b]], lv.at[b, 0], gsem
        ).start()
        return 0

    lax.fori_loop(0, B, fire, 0)

    def body(i, _):
        b = i * _RP
        j = lax.rem(i, _NBUF)

        @pl.when(i >= _NBUF)
        def _():
            pltpu.make_async_copy(
                rep.at[j], out_ref.at[pl.ds((i - _NBUF) * _RP, _RP)], sems.at[j]
            ).wait()

        for k in range(_RP):
            pltpu.make_async_copy(
                vals_ref.at[b + k, 0], lv.at[b + k, 0], gsem
            ).wait()
            row = lv[pl.ds(b + k, 1), 0, :]
            rep[pl.ds(j, 1), pl.ds(k, 1)] = jnp.broadcast_to(
                row[None, None], (1, 1, L, D)
            )
        pltpu.make_async_copy(
            rep.at[j], out_ref.at[pl.ds(b, _RP)], sems.at[j]
        ).start()
        return 0

    n = B // _RP
    lax.fori_loop(0, n, body, 0)

    def drain(t, _):
        i = n - _NBUF + t
        pltpu.make_async_copy(
            rep.at[lax.rem(i, _NBUF)],
            out_ref.at[pl.ds(i * _RP, _RP)],
            sems.at[lax.rem(i, _NBUF)],
        ).wait()
        return 0

    lax.fori_loop(0, _NBUF, drain, 0)


def kernel(input_values, input_timestamps, is_target_mask, dummy):
    B, L, D = input_values.shape
    mask_i8 = is_target_mask.view(jnp.int8)

    out = pl.pallas_call(
        _fused_kernel,
        in_specs=[
            pl.BlockSpec(memory_space=pltpu.VMEM),
            pl.BlockSpec(memory_space=pl.ANY),
        ],
        out_specs=pl.BlockSpec(memory_space=pl.ANY),
        scratch_shapes=[
            pltpu.VMEM((B,), jnp.int32),
            pltpu.SMEM((B,), jnp.int32),
            pltpu.VMEM((B, 1, D), jnp.float32),
            pltpu.VMEM((_NBUF, _RP, L, D), jnp.float32),
            pltpu.SemaphoreType.DMA,
            pltpu.SemaphoreType.DMA,
            pltpu.SemaphoreType.DMA((_NBUF,)),
        ],
        out_shape=jax.ShapeDtypeStruct((B, L, D), jnp.float32),
    )(mask_i8, input_values)
    return out


# speedup vs baseline: 1.6433x; 1.0071x over previous
import jax
import jax.numpy as jnp
from jax import lax
from jax.experimental import pallas as pl
from jax.experimental.pallas import tpu as pltpu

_NBUF = 4


def _fused_kernel(mask_ref, vals_ref, out_ref, idx_vmem, idx_smem, lv, rep,
                  isem, gsem, sems):
    B, L = mask_ref.shape
    D = lv.shape[-1]

    pos = lax.broadcasted_iota(jnp.int32, (B, L), 1)
    m = mask_ref[...].astype(jnp.int32)
    cand = jnp.where(m == 0, pos, -1)
    idx_vmem[...] = jnp.maximum(jnp.max(cand, axis=1), 0)
    cp = pltpu.make_async_copy(idx_vmem, idx_smem, isem)
    cp.start()
    cp.wait()

    def fire(b, _):
        pltpu.make_async_copy(
            vals_ref.at[b, idx_smem[b]], lv.at[b, 0], gsem
        ).start()
        return 0

    lax.fori_loop(0, B, fire, 0)

    def body(b, _):
        pltpu.make_async_copy(vals_ref.at[b, 0], lv.at[b, 0], gsem).wait()
        j = lax.rem(b, _NBUF)

        @pl.when(b >= _NBUF)
        def _():
            pltpu.make_async_copy(
                rep.at[j], out_ref.at[b - _NBUF], sems.at[j]
            ).wait()

        row = lv[pl.ds(b, 1), 0, :]
        rep[pl.ds(j, 1)] = jnp.broadcast_to(row[None], (1, L, D))
        pltpu.make_async_copy(rep.at[j], out_ref.at[b], sems.at[j]).start()
        return 0

    lax.fori_loop(0, B, body, 0)

    def drain(k, _):
        b = B - _NBUF + k
        pltpu.make_async_copy(
            rep.at[lax.rem(b, _NBUF)], out_ref.at[b], sems.at[lax.rem(b, _NBUF)]
        ).wait()
        return 0

    lax.fori_loop(0, _NBUF, drain, 0)


def kernel(input_values, input_timestamps, is_target_mask, dummy):
    B, L, D = input_values.shape
    mask_i8 = is_target_mask.view(jnp.int8)

    out = pl.pallas_call(
        _fused_kernel,
        in_specs=[
            pl.BlockSpec(memory_space=pltpu.VMEM),
            pl.BlockSpec(memory_space=pl.ANY),
        ],
        out_specs=pl.BlockSpec(memory_space=pl.ANY),
        scratch_shapes=[
            pltpu.VMEM((B,), jnp.int32),
            pltpu.SMEM((B,), jnp.int32),
            pltpu.VMEM((B, 1, D), jnp.float32),
            pltpu.VMEM((_NBUF, L, D), jnp.float32),
            pltpu.SemaphoreType.DMA,
            pltpu.SemaphoreType.DMA,
            pltpu.SemaphoreType.DMA((_NBUF,)),
        ],
        out_shape=jax.ShapeDtypeStruct((B, L, D), jnp.float32),
    )(mask_i8, input_values)
    return out
